# Initial kernel scaffold; baseline (speedup 1.0000x reference)
import jax
import jax.numpy as jnp
from jax.experimental import pallas as pl


def kernel(input, W_main, W_mask):
    raise NotImplementedError("write your pallas kernel here")


# speedup vs baseline: 1.7542x; 1.7542x over previous
import functools

import jax
import jax.numpy as jnp
from jax import lax
from jax.experimental import pallas as pl
from jax.experimental.pallas import tpu as pltpu
from jax.experimental.pallas import tpu_sc as plsc

MASK_LO = 900000
DIM = 64
CHUNK = 512


def _sc_embed(idx, W_main, W_mask, *, interpret=False):
    N = idx.shape[0]
    info = plsc.get_sparse_core_info()
    NC, NS, L = info.num_cores, info.num_subcores, info.num_lanes
    NW = NC * NS
    assert N % (NW * CHUNK) == 0
    per_w = N // NW
    n_chunks = per_w // CHUNK
    n_vecs = CHUNK // L

    mesh = plsc.VectorSubcoreMesh(core_axis_name="c", subcore_axis_name="s")

    @functools.partial(
        pl.kernel,
        out_type=jax.ShapeDtypeStruct((N, DIM), jnp.float32),
        mesh=mesh,
        scratch_types=[
            pltpu.VMEM((CHUNK,), jnp.int32),
            pltpu.VMEM((CHUNK,), jnp.int32),
            pltpu.VMEM((CHUNK,), jnp.int32),
            pltpu.VMEM((CHUNK, DIM), jnp.float32),
            pltpu.VMEM((CHUNK, DIM), jnp.float32),
            pltpu.SemaphoreType.DMA,
            pltpu.SemaphoreType.DMA,
            pltpu.SemaphoreType.DMA,
        ],
        compiler_params=pltpu.CompilerParams(use_tc_tiling_on_sc=False),
        interpret=interpret,
    )
    def k(idx_hbm, wmain_hbm, wmask_hbm, out_hbm,
          idx_v, midx_v, gpos_v, rows_v, mrows_v, sem_a, sem_b, sem_c):
        wid = lax.axis_index("s") * NC + lax.axis_index("c")
        wbase = wid * per_w
        lane = lax.iota(jnp.int32, L)

        def chunk_body(c, _):
            cbase = wbase + c * CHUNK
            pltpu.sync_copy(idx_hbm.at[pl.ds(cbase, CHUNK)], idx_v)
            g_main = pltpu.async_copy(wmain_hbm.at[idx_v], rows_v, sem_a)

            def scan_vec(i, m_carry):
                v = idx_v[pl.ds(i * L, L)]
                is_m = v >= MASK_LO
                key = jnp.where(is_m, ((i * L + lane) << 17) | (v - MASK_LO),
                                -1)
                return jnp.maximum(m_carry, key)

            M = lax.fori_loop(0, n_vecs, scan_vec,
                              jnp.full((L,), -1, jnp.int32))
            for sh in (8, 4, 2, 1):
                rot = ((lane + sh) & (L - 1)).astype(jnp.int32)
                M = jnp.maximum(M, M.at[rot].get(mode="promise_in_bounds"))
            has_mask = M[0] >= 0
            Mc = jnp.maximum(M, 0)
            fm_pos = Mc >> 17
            fm_midx = Mc & 0x1FFFF

            def fix_vec(i, _):
                v = idx_v[pl.ds(i * L, L)]
                is_m = v >= MASK_LO
                midx_v[pl.ds(i * L, L)] = jnp.where(is_m, v - MASK_LO,
                                                    fm_midx)
                gpos_v[pl.ds(i * L, L)] = cbase + jnp.where(
                    is_m, i * L + lane, fm_pos)
                return 0

            lax.fori_loop(0, n_vecs, fix_vec, 0)

            g_mask = pltpu.async_copy(wmask_hbm.at[midx_v], mrows_v, sem_b)
            g_main.wait()
            pltpu.async_copy(rows_v, out_hbm.at[pl.ds(cbase, CHUNK)],
                             sem_c).wait()
            g_mask.wait()

            @pl.when(has_mask)
            def _():
                pltpu.async_copy(mrows_v, out_hbm.at[gpos_v], sem_b).wait()

            return 0

        lax.fori_loop(0, n_chunks, chunk_body, 0)

    return k(idx, W_main, W_mask)


def kernel(input, W_main, W_mask):
    B, H = input.shape
    out = _sc_embed(input.reshape(B * H), W_main, W_mask)
    return out.reshape(B, H, DIM)
